# baseline (speedup 1.0000x reference)
import jax
import jax.numpy as jnp
from jax.experimental import pallas as pl

_N_BUCKETS = 1024.0
_BANDWIDTH = 1.0

_ROWS = 2000


def _lsh_block(x_ref, rv_ref, out_ref):
    proj = jnp.dot(x_ref[...], rv_ref[...], preferred_element_type=jnp.float32)
    out_ref[...] = jnp.floor(proj / _BANDWIDTH) % _N_BUCKETS


def kernel(x, random_vectors):
    n, d = x.shape
    h = random_vectors.shape[1]
    grid = (n // _ROWS,)
    return pl.pallas_call(
        _lsh_block,
        grid=grid,
        in_specs=[
            pl.BlockSpec((_ROWS, d), lambda i: (i, 0)),
            pl.BlockSpec((d, h), lambda i: (0, 0)),
        ],
        out_specs=pl.BlockSpec((_ROWS, h), lambda i: (i, 0)),
        out_shape=jax.ShapeDtypeStruct((n, h), jnp.float32),
    )(x, random_vectors)


# speedup vs baseline: 1.0621x; 1.0621x over previous
import jax
import jax.numpy as jnp
from jax.experimental import pallas as pl

_N_BUCKETS = 1024.0
_BANDWIDTH = 1.0

_ROWS = 2000


def _lsh_block(x_ref, rv_ref, out_ref):
    proj = jnp.dot(x_ref[...], rv_ref[...], preferred_element_type=jnp.float32)
    i = jnp.floor(proj / _BANDWIDTH).astype(jnp.int32)
    out_ref[...] = (i & 1023).astype(jnp.float32)


def kernel(x, random_vectors):
    n, d = x.shape
    h = random_vectors.shape[1]
    grid = (n // _ROWS,)
    return pl.pallas_call(
        _lsh_block,
        grid=grid,
        in_specs=[
            pl.BlockSpec((_ROWS, d), lambda i: (i, 0)),
            pl.BlockSpec((d, h), lambda i: (0, 0)),
        ],
        out_specs=pl.BlockSpec((_ROWS, h), lambda i: (i, 0)),
        out_shape=jax.ShapeDtypeStruct((n, h), jnp.float32),
    )(x, random_vectors)


# speedup vs baseline: 1.3944x; 1.3128x over previous
import jax
import jax.numpy as jnp
from jax.experimental import pallas as pl

_N_BUCKETS = 1024.0
_BANDWIDTH = 1.0

_ROWS = 10000


def _lsh_block(x_ref, rv_ref, out_ref):
    proj = jnp.dot(x_ref[...], rv_ref[...], preferred_element_type=jnp.float32)
    i = jnp.floor(proj / _BANDWIDTH).astype(jnp.int32)
    out_ref[...] = (i & 1023).astype(jnp.float32)


def kernel(x, random_vectors):
    n, d = x.shape
    h = random_vectors.shape[1]
    grid = (n // _ROWS,)
    return pl.pallas_call(
        _lsh_block,
        grid=grid,
        in_specs=[
            pl.BlockSpec((_ROWS, d), lambda i: (i, 0)),
            pl.BlockSpec((d, h), lambda i: (0, 0)),
        ],
        out_specs=pl.BlockSpec((_ROWS, h), lambda i: (i, 0)),
        out_shape=jax.ShapeDtypeStruct((n, h), jnp.float32),
    )(x, random_vectors)
